# baseline (speedup 1.0000x reference)
import functools

import jax
import jax.numpy as jnp
from jax import lax
from jax.experimental import pallas as pl
from jax.experimental.pallas import tpu as pltpu
from jax.experimental.pallas import tpu_sc as plsc

_B, _S, _D = 4096, 200, 64
_N = _B * _S
_TBL = 1024

_info = plsc.get_sparse_core_info()
_NC, _NS, _L = _info.num_cores, _info.num_subcores, _info.num_lanes
_NW = _NC * _NS
_BPW = _B // _NW
_PER_W = _BPW * _S
_DT = _D // 8
_SLOTS = 4
_OUTER = _S // _SLOTS

_mesh = plsc.VectorSubcoreMesh(core_axis_name="c", subcore_axis_name="s")


@functools.partial(
    pl.kernel,
    mesh=_mesh,
    out_type=jax.ShapeDtypeStruct((_S, _DT, _NW, 8 * _BPW), jnp.float32),
    compiler_params=pltpu.CompilerParams(
        use_tc_tiling_on_sc=False, needs_layout_passes=False),
    scratch_types=[
        pltpu.VMEM((2, 16), jnp.float32),
        pltpu.VMEM((_PER_W,), jnp.int32),
        pltpu.VMEM((_SLOTS, _BPW), jnp.int32),
        pltpu.VMEM((_SLOTS, _BPW), jnp.int32),
        pltpu.VMEM((_SLOTS, _BPW, _D), jnp.float32),
        pltpu.VMEM((_SLOTS, _BPW, _D), jnp.float32),
        pltpu.VMEM((_SLOTS, _D * _BPW), jnp.float32),
        pltpu.SemaphoreType.DMA,
        pltpu.SemaphoreType.DMA,
        pltpu.SemaphoreType.DMA,
        pltpu.SemaphoreType.DMA,
        pltpu.SemaphoreType.DMA,
        pltpu.SemaphoreType.DMA,
        pltpu.SemaphoreType.DMA,
        pltpu.SemaphoreType.DMA,
    ],
)
def _sc_fused_lookup(items_hbm, table_hbm, w_hbm, out_hbm,
                     w_v, item_all, idx0_v, idx1_v, rowsA, rowsB, out_v,
                     gs0, gs1, gs2, gs3, ow0, ow1, ow2, ow3):
    gs = (gs0, gs1, gs2, gs3)
    ow = (ow0, ow1, ow2, ow3)
    wid = lax.axis_index("s") * _NC + lax.axis_index("c")

    pltpu.sync_copy(w_hbm, w_v)
    e0 = jnp.exp(w_v[0, :])
    e1 = jnp.exp(w_v[1, :])
    w0 = e0 / (e0 + e1)
    w1 = e1 / (e0 + e1)

    pltpu.sync_copy(items_hbm.at[pl.ds(wid * _PER_W, _PER_W)], item_all)

    lane = lax.iota(jnp.int32, _L)
    lane200 = lane * _S
    bases = [lane * _BPW + 2048 * j for j in range(_D // _L)]

    def compute_idx(c, s):
        d0 = idx0_v.at[s]
        d1 = idx1_v.at[s]
        for j in range(_BPW // _L):
            v = plsc.load_gather(item_all, [lane200 + (j * _L * _S + c)])
            sl = pl.ds(j * _L, _L)
            d0[sl] = v & (_TBL - 1)
            d1[sl] = ((v >> 10) & (_TBL - 1)) + _TBL

    def issue_gathers(s):
        pltpu.async_copy(table_hbm.at[idx0_v.at[s]], rowsA.at[s], gs[s])
        pltpu.async_copy(table_hbm.at[idx1_v.at[s]], rowsB.at[s], gs[s])

    def wait_gathers(s):
        pltpu.make_async_copy(table_hbm.at[idx0_v.at[s]], rowsA.at[s], gs[s]).wait()
        pltpu.make_async_copy(table_hbm.at[idx1_v.at[s]], rowsB.at[s], gs[s]).wait()

    def issue_out(c, s):
        for dt in range(_DT):
            pltpu.async_copy(out_v.at[s, pl.ds(dt * 8 * _BPW, 8 * _BPW)],
                             out_hbm.at[c, dt, wid], ow[s])

    def wait_out(s):
        for dt in range(_DT):
            pltpu.make_async_copy(out_v.at[s, pl.ds(dt * 8 * _BPW, 8 * _BPW)],
                                  out_hbm.at[0, dt, 0], ow[s]).wait()

    for s in range(_SLOTS):
        compute_idx(s, s)
        issue_gathers(s)

    def outer(i, carry):
        for s in range(_SLOTS):
            c = i * _SLOTS + s
            wait_gathers(s)

            @pl.when(i > 0)
            def _():
                wait_out(s)

            def row_body(r, rvec):
                a = rowsA.at[s].at[r]
                b = rowsB.at[s].at[r]
                for j in range(_D // _L):
                    sl = pl.ds(j * _L, _L)
                    v = a[sl] * w0 + b[sl] * w1
                    plsc.store_scatter(out_v.at[s], [bases[j] + rvec], v)
                return rvec + 1

            lax.fori_loop(0, _BPW, row_body,
                          jnp.zeros((_L,), jnp.int32), unroll=8)

            issue_out(c, s)

            @pl.when(i < _OUTER - 1)
            def _():
                compute_idx(c + _SLOTS, s)
                issue_gathers(s)

        return carry

    lax.fori_loop(0, _OUTER, outer, 0)

    for s in range(_SLOTS):
        wait_out(s)


def kernel(item_seq, tables, fusion_weights):
    items_flat = item_seq.reshape(_N)
    table2d = tables.reshape(2 * _TBL, _D)
    w_pad = jnp.broadcast_to(fusion_weights.reshape(2, 1), (2, 16))
    out4 = _sc_fused_lookup(items_flat, table2d, w_pad)
    out5 = out4.reshape(_S, _DT, _NW, 8, _BPW)
    return out5.transpose(2, 4, 0, 1, 3).reshape(_B, _S, _D)


# speedup vs baseline: 1.6346x; 1.6346x over previous
import functools

import jax
import jax.numpy as jnp
from jax import lax
from jax.experimental import pallas as pl
from jax.experimental.pallas import tpu as pltpu
from jax.experimental.pallas import tpu_sc as plsc

_B, _S, _D = 4096, 200, 64
_N = _B * _S
_TBL = 1024

_info = plsc.get_sparse_core_info()
_NC, _NS, _L = _info.num_cores, _info.num_subcores, _info.num_lanes
_NW = _NC * _NS
_BPW = _B // _NW
_PER_W = _BPW * _S
_DT = _D // 8
_SLOTS = 4
_OUTER = _S // _SLOTS

_mesh = plsc.VectorSubcoreMesh(core_axis_name="c", subcore_axis_name="s")


@functools.partial(
    pl.kernel,
    mesh=_mesh,
    out_type=jax.ShapeDtypeStruct((_S, _DT, _NW, 8 * _BPW), jnp.float32),
    compiler_params=pltpu.CompilerParams(
        use_tc_tiling_on_sc=False, needs_layout_passes=False),
    scratch_types=[
        pltpu.VMEM((2, 16), jnp.float32),
        pltpu.VMEM((_PER_W,), jnp.int32),
        pltpu.VMEM((_SLOTS, _BPW), jnp.int32),
        pltpu.VMEM((_SLOTS, _BPW), jnp.int32),
        pltpu.VMEM((_SLOTS, _BPW, _D), jnp.float32),
        pltpu.VMEM((_SLOTS, _BPW, _D), jnp.float32),
        pltpu.VMEM((_SLOTS, _D * _BPW), jnp.float32),
        pltpu.SemaphoreType.DMA,
        pltpu.SemaphoreType.DMA,
        pltpu.SemaphoreType.DMA,
        pltpu.SemaphoreType.DMA,
        pltpu.SemaphoreType.DMA,
        pltpu.SemaphoreType.DMA,
        pltpu.SemaphoreType.DMA,
        pltpu.SemaphoreType.DMA,
    ],
)
def _sc_fused_lookup(items_hbm, table_hbm, w_hbm, out_hbm,
                     w_v, item_all, idx0_v, idx1_v, rowsA, rowsB, out_v,
                     gs0, gs1, gs2, gs3, ow0, ow1, ow2, ow3):
    gs = (gs0, gs1, gs2, gs3)
    ow = (ow0, ow1, ow2, ow3)
    wid = lax.axis_index("s") * _NC + lax.axis_index("c")

    pltpu.sync_copy(w_hbm, w_v)
    e0 = jnp.exp(w_v[0, :])
    e1 = jnp.exp(w_v[1, :])
    w0 = e0 / (e0 + e1)
    w1 = e1 / (e0 + e1)

    pltpu.sync_copy(items_hbm.at[pl.ds(wid * _PER_W, _PER_W)], item_all)

    lane = lax.iota(jnp.int32, _L)
    lane200 = lane * _S
    bases = [lane * _BPW + 2048 * j for j in range(_D // _L)]

    def compute_idx(c, s):
        d0 = idx0_v.at[s]
        d1 = idx1_v.at[s]
        for j in range(_BPW // _L):
            v = plsc.load_gather(item_all, [lane200 + (j * _L * _S + c)])
            sl = pl.ds(j * _L, _L)
            d0[sl] = v & (_TBL - 1)
            d1[sl] = ((v >> 10) & (_TBL - 1)) + _TBL

    def issue_gathers(s):
        pltpu.async_copy(table_hbm.at[idx0_v.at[s]], rowsA.at[s], gs[s])
        pltpu.async_copy(table_hbm.at[idx1_v.at[s]], rowsB.at[s], gs[s])

    def wait_gathers(s):
        pltpu.make_async_copy(table_hbm.at[idx0_v.at[s]], rowsA.at[s], gs[s]).wait()
        pltpu.make_async_copy(table_hbm.at[idx1_v.at[s]], rowsB.at[s], gs[s]).wait()

    def issue_out(c, s):
        for dt in range(_DT):
            pltpu.async_copy(out_v.at[s, pl.ds(dt * 8 * _BPW, 8 * _BPW)],
                             out_hbm.at[c, dt, wid], ow[s])

    def wait_out(s):
        for dt in range(_DT):
            pltpu.make_async_copy(out_v.at[s, pl.ds(dt * 8 * _BPW, 8 * _BPW)],
                                  out_hbm.at[0, dt, 0], ow[s]).wait()

    for s in range(_SLOTS):
        compute_idx(s, s)
        issue_gathers(s)

    def outer(i, carry):
        for s in range(_SLOTS):
            c = i * _SLOTS + s
            wait_gathers(s)

            @pl.when(i > 0)
            def _():
                wait_out(s)

            def row_body(r, rvec):
                a = rowsA.at[s].at[r]
                b = rowsB.at[s].at[r]
                for j in range(_D // _L):
                    sl = pl.ds(j * _L, _L)
                    v = a[sl] * w0 + b[sl] * w1
                    plsc.store_scatter(out_v.at[s], [bases[j] + rvec], v)
                return rvec + 1

            plsc.parallel_loop(0, _BPW, 1, unroll=4,
                               carry=jnp.zeros((_L,), jnp.int32))(row_body)

            issue_out(c, s)

            @pl.when(i < _OUTER - 1)
            def _():
                compute_idx(c + _SLOTS, s)
                issue_gathers(s)

        return carry

    lax.fori_loop(0, _OUTER, outer, 0)

    for s in range(_SLOTS):
        wait_out(s)


def kernel(item_seq, tables, fusion_weights):
    items_flat = item_seq.reshape(_N)
    table2d = tables.reshape(2 * _TBL, _D)
    w_pad = jnp.broadcast_to(fusion_weights.reshape(2, 1), (2, 16))
    out4 = _sc_fused_lookup(items_flat, table2d, w_pad)
    out5 = out4.reshape(_S, _DT, _NW, 8, _BPW)
    return out5.transpose(2, 4, 0, 1, 3).reshape(_B, _S, _D)


# speedup vs baseline: 1.6351x; 1.0003x over previous
import functools

import jax
import jax.numpy as jnp
from jax import lax
from jax.experimental import pallas as pl
from jax.experimental.pallas import tpu as pltpu
from jax.experimental.pallas import tpu_sc as plsc

_B, _S, _D = 4096, 200, 64
_N = _B * _S
_TBL = 1024

_info = plsc.get_sparse_core_info()
_NC, _NS, _L = _info.num_cores, _info.num_subcores, _info.num_lanes
_NW = _NC * _NS
_BPW = _B // _NW
_PER_W = _BPW * _S
_DT = _D // 8
_SLOTS = 4
_OUTER = _S // _SLOTS

_mesh = plsc.VectorSubcoreMesh(core_axis_name="c", subcore_axis_name="s")


@functools.partial(
    pl.kernel,
    mesh=_mesh,
    out_type=jax.ShapeDtypeStruct((_S, _DT, _NW, 8 * _BPW), jnp.float32),
    compiler_params=pltpu.CompilerParams(
        use_tc_tiling_on_sc=False, needs_layout_passes=False),
    scratch_types=[
        pltpu.VMEM((2, 16), jnp.float32),
        pltpu.VMEM((_PER_W,), jnp.int32),
        pltpu.VMEM((_SLOTS, _BPW), jnp.int32),
        pltpu.VMEM((_SLOTS, _BPW), jnp.int32),
        pltpu.VMEM((_SLOTS, _BPW, _D), jnp.float32),
        pltpu.VMEM((_SLOTS, _BPW, _D), jnp.float32),
        pltpu.VMEM((_SLOTS, _D * _BPW), jnp.float32),
        pltpu.SemaphoreType.DMA,
        pltpu.SemaphoreType.DMA,
        pltpu.SemaphoreType.DMA,
        pltpu.SemaphoreType.DMA,
        pltpu.SemaphoreType.DMA,
        pltpu.SemaphoreType.DMA,
        pltpu.SemaphoreType.DMA,
        pltpu.SemaphoreType.DMA,
    ],
)
def _sc_fused_lookup(items_hbm, table_hbm, w_hbm, out_hbm,
                     w_v, item_all, idx0_v, idx1_v, rowsA, rowsB, out_v,
                     gs0, gs1, gs2, gs3, ow0, ow1, ow2, ow3):
    gs = (gs0, gs1, gs2, gs3)
    ow = (ow0, ow1, ow2, ow3)
    wid = lax.axis_index("s") * _NC + lax.axis_index("c")

    pltpu.sync_copy(w_hbm, w_v)
    e0 = jnp.exp(w_v[0, :])
    e1 = jnp.exp(w_v[1, :])
    w0 = e0 / (e0 + e1)
    w1 = e1 / (e0 + e1)

    pltpu.sync_copy(items_hbm.at[pl.ds(wid * _PER_W, _PER_W)], item_all)

    lane = lax.iota(jnp.int32, _L)
    lane200 = lane * _S
    bases = [lane * _BPW + 2048 * j for j in range(_D // _L)]

    def compute_idx(c, s):
        d0 = idx0_v.at[s]
        d1 = idx1_v.at[s]
        for j in range(_BPW // _L):
            v = plsc.load_gather(item_all, [lane200 + (j * _L * _S + c)])
            sl = pl.ds(j * _L, _L)
            d0[sl] = v & (_TBL - 1)
            d1[sl] = ((v >> 10) & (_TBL - 1)) + _TBL

    def issue_gathers(s):
        pltpu.async_copy(table_hbm.at[idx0_v.at[s]], rowsA.at[s], gs[s])
        pltpu.async_copy(table_hbm.at[idx1_v.at[s]], rowsB.at[s], gs[s])

    def wait_gathers(s):
        pltpu.make_async_copy(table_hbm.at[idx0_v.at[s]], rowsA.at[s], gs[s]).wait()
        pltpu.make_async_copy(table_hbm.at[idx1_v.at[s]], rowsB.at[s], gs[s]).wait()

    def issue_out(c, s):
        for dt in range(_DT):
            pltpu.async_copy(out_v.at[s, pl.ds(dt * 8 * _BPW, 8 * _BPW)],
                             out_hbm.at[c, dt, wid], ow[s])

    def wait_out(s):
        for dt in range(_DT):
            pltpu.make_async_copy(out_v.at[s, pl.ds(dt * 8 * _BPW, 8 * _BPW)],
                                  out_hbm.at[0, dt, 0], ow[s]).wait()

    for s in range(_SLOTS):
        compute_idx(s, s)
        issue_gathers(s)

    def outer(i, carry):
        for s in range(_SLOTS):
            c = i * _SLOTS + s
            wait_gathers(s)

            @pl.when(i > 0)
            def _():
                wait_out(s)

            def row_body(r, rvec):
                a = rowsA.at[s].at[r]
                b = rowsB.at[s].at[r]
                for j in range(_D // _L):
                    sl = pl.ds(j * _L, _L)
                    v = a[sl] * w0 + b[sl] * w1
                    plsc.store_scatter(out_v.at[s], [bases[j] + rvec], v)
                return rvec + 1

            plsc.parallel_loop(0, _BPW, 1, unroll=2,
                               carry=jnp.zeros((_L,), jnp.int32))(row_body)

            issue_out(c, s)

            @pl.when(i < _OUTER - 1)
            def _():
                compute_idx(c + _SLOTS, s)
                issue_gathers(s)

        return carry

    lax.fori_loop(0, _OUTER, outer, 0)

    for s in range(_SLOTS):
        wait_out(s)


def kernel(item_seq, tables, fusion_weights):
    items_flat = item_seq.reshape(_N)
    table2d = tables.reshape(2 * _TBL, _D)
    w_pad = jnp.broadcast_to(fusion_weights.reshape(2, 1), (2, 16))
    out4 = _sc_fused_lookup(items_flat, table2d, w_pad)
    out5 = out4.reshape(_S, _DT, _NW, 8, _BPW)
    return out5.transpose(2, 4, 0, 1, 3).reshape(_B, _S, _D)


# speedup vs baseline: 1.6360x; 1.0006x over previous
import functools

import jax
import jax.numpy as jnp
from jax import lax
from jax.experimental import pallas as pl
from jax.experimental.pallas import tpu as pltpu
from jax.experimental.pallas import tpu_sc as plsc

_B, _S, _D = 4096, 200, 64
_N = _B * _S
_TBL = 1024

_info = plsc.get_sparse_core_info()
_NC, _NS, _L = _info.num_cores, _info.num_subcores, _info.num_lanes
_NW = _NC * _NS
_BPW = _B // _NW
_PER_W = _BPW * _S
_DT = _D // 8
_SLOTS = 4
_OUTER = _S // _SLOTS

_mesh = plsc.VectorSubcoreMesh(core_axis_name="c", subcore_axis_name="s")


@functools.partial(
    pl.kernel,
    mesh=_mesh,
    out_type=jax.ShapeDtypeStruct((_S, _DT, _NW, 8 * _BPW), jnp.float32),
    compiler_params=pltpu.CompilerParams(
        use_tc_tiling_on_sc=False, needs_layout_passes=False),
    scratch_types=[
        pltpu.VMEM((2, 16), jnp.float32),
        pltpu.VMEM((_PER_W,), jnp.int32),
        pltpu.VMEM((_SLOTS, _BPW), jnp.int32),
        pltpu.VMEM((_SLOTS, _BPW), jnp.int32),
        pltpu.VMEM((_SLOTS, _BPW, _D), jnp.float32),
        pltpu.VMEM((_SLOTS, _BPW, _D), jnp.float32),
        pltpu.VMEM((_SLOTS, _D * _BPW), jnp.float32),
        pltpu.SemaphoreType.DMA,
        pltpu.SemaphoreType.DMA,
        pltpu.SemaphoreType.DMA,
        pltpu.SemaphoreType.DMA,
        pltpu.SemaphoreType.DMA,
        pltpu.SemaphoreType.DMA,
        pltpu.SemaphoreType.DMA,
        pltpu.SemaphoreType.DMA,
    ],
)
def _sc_fused_lookup(items_hbm, table_hbm, w_hbm, out_hbm,
                     w_v, item_all, idx0_v, idx1_v, rowsA, rowsB, out_v,
                     gs0, gs1, gs2, gs3, ow0, ow1, ow2, ow3):
    gs = (gs0, gs1, gs2, gs3)
    ow = (ow0, ow1, ow2, ow3)
    wid = lax.axis_index("s") * _NC + lax.axis_index("c")

    pltpu.sync_copy(w_hbm, w_v)
    e0 = jnp.exp(w_v[0, :])
    e1 = jnp.exp(w_v[1, :])
    w0 = e0 / (e0 + e1)
    w1 = e1 / (e0 + e1)

    pltpu.sync_copy(items_hbm.at[pl.ds(wid * _PER_W, _PER_W)], item_all)

    lane = lax.iota(jnp.int32, _L)
    lane200 = lane * _S
    bases = [lane * _BPW + 2048 * j for j in range(_D // _L)]

    def compute_idx(c, s):
        d0 = idx0_v.at[s]
        d1 = idx1_v.at[s]
        for j in range(_BPW // _L):
            v = plsc.load_gather(item_all, [lane200 + (j * _L * _S + c)])
            sl = pl.ds(j * _L, _L)
            d0[sl] = v & (_TBL - 1)
            d1[sl] = ((v >> 10) & (_TBL - 1)) + _TBL

    def issue_gathers(s):
        pltpu.async_copy(table_hbm.at[idx0_v.at[s]], rowsA.at[s], gs[s])
        pltpu.async_copy(table_hbm.at[idx1_v.at[s]], rowsB.at[s], gs[s])

    def wait_gathers(s):
        pltpu.make_async_copy(table_hbm.at[idx0_v.at[s]], rowsA.at[s], gs[s]).wait()
        pltpu.make_async_copy(table_hbm.at[idx1_v.at[s]], rowsB.at[s], gs[s]).wait()

    def issue_out(c, s):
        for dt in range(_DT):
            pltpu.async_copy(out_v.at[s, pl.ds(dt * 8 * _BPW, 8 * _BPW)],
                             out_hbm.at[c, dt, wid], ow[s])

    def wait_out(s):
        for dt in range(_DT):
            pltpu.make_async_copy(out_v.at[s, pl.ds(dt * 8 * _BPW, 8 * _BPW)],
                                  out_hbm.at[0, dt, 0], ow[s]).wait()

    for s in range(_SLOTS):
        compute_idx(s, s)
        issue_gathers(s)

    def outer(i, carry):
        for s in range(_SLOTS):
            c = i * _SLOTS + s
            wait_gathers(s)

            @pl.when(i > 0)
            def _():
                wait_out(s)

            def row_body(r, rvec):
                a = rowsA.at[s].at[r]
                b = rowsB.at[s].at[r]
                for j in range(_D // _L):
                    sl = pl.ds(j * _L, _L)
                    v = a[sl] * w0 + b[sl] * w1
                    plsc.store_scatter(out_v.at[s], [bases[j] + rvec], v)
                return rvec + 1

            plsc.parallel_loop(0, _BPW, 1, unroll=4,
                               carry=jnp.zeros((_L,), jnp.int32))(row_body)

            issue_out(c, s)

            @pl.when(i < _OUTER - 1)
            def _():
                compute_idx(c + _SLOTS, s)
                issue_gathers(s)

        return carry

    lax.fori_loop(0, _OUTER, outer, 0)

    for s in range(_SLOTS):
        wait_out(s)


def kernel(item_seq, tables, fusion_weights):
    items_flat = item_seq.reshape(_N)
    table2d = tables.reshape(2 * _TBL, _D)
    w_pad = jnp.broadcast_to(fusion_weights.reshape(2, 1), (2, 16))
    out4 = _sc_fused_lookup(items_flat, table2d, w_pad)
    out5 = out4.reshape(_S, _DT, _NW, 8, _BPW)
    return out5.transpose(2, 4, 0, 1, 3).reshape(_B, _S, _D)
